# baseline (speedup 1.0000x reference)
import functools

import jax
import jax.numpy as jnp
from jax.experimental import pallas as pl
from jax.experimental.pallas import tpu as pltpu


def _pool_kernel(x_ref, o_ref, *, c):
    x = x_ref[0]
    s = x[:, 0] + x[:, 1]
    o_ref[0] = (s[..., :c] + s[..., c:]) * 0.5


def kernel(inputs):
    b, h, w, c = inputs.shape
    h2, w2 = h // 2, w // 2
    view = inputs.reshape(b, h2, 2, w2, 2 * c)

    hb = 32
    while h2 % hb:
        hb //= 2

    return pl.pallas_call(
        functools.partial(_pool_kernel, c=c),
        grid=(b, h2 // hb),
        in_specs=[
            pl.BlockSpec((1, hb, 2, w2, 2 * c), lambda bi, hi: (bi, hi, 0, 0, 0)),
        ],
        out_specs=pl.BlockSpec((1, hb, w2, c), lambda bi, hi: (bi, hi, 0, 0)),
        out_shape=jax.ShapeDtypeStruct((b, h2, w2, c), inputs.dtype),
        compiler_params=pltpu.CompilerParams(
            dimension_semantics=("parallel", "arbitrary"),
        ),
    )(view)


# speedup vs baseline: 1.2196x; 1.2196x over previous
import functools

import jax
import jax.numpy as jnp
from jax.experimental import pallas as pl
from jax.experimental.pallas import tpu as pltpu


def _pool_kernel(x_ref, o_ref, *, hb):
    w2 = o_ref.shape[2]
    a = x_ref[0, pl.ds(0, hb, 2), pl.ds(0, w2, 2), :]
    b = x_ref[0, pl.ds(0, hb, 2), pl.ds(1, w2, 2), :]
    c = x_ref[0, pl.ds(1, hb, 2), pl.ds(0, w2, 2), :]
    d = x_ref[0, pl.ds(1, hb, 2), pl.ds(1, w2, 2), :]
    o_ref[0] = (a + b + c + d) * 0.5


def kernel(inputs):
    b, h, w, c = inputs.shape
    h2, w2 = h // 2, w // 2

    hb = 16
    while h2 % hb:
        hb //= 2

    return pl.pallas_call(
        functools.partial(_pool_kernel, hb=hb),
        grid=(b, h2 // hb),
        in_specs=[
            pl.BlockSpec((1, 2 * hb, w, c), lambda bi, hi: (bi, hi, 0, 0)),
        ],
        out_specs=pl.BlockSpec((1, hb, w2, c), lambda bi, hi: (bi, hi, 0, 0)),
        out_shape=jax.ShapeDtypeStruct((b, h2, w2, c), inputs.dtype),
        compiler_params=pltpu.CompilerParams(
            dimension_semantics=(pltpu.PARALLEL, pltpu.ARBITRARY),
        ),
    )(inputs)


# speedup vs baseline: 1.2204x; 1.0007x over previous
import functools

import jax
import jax.numpy as jnp
from jax.experimental import pallas as pl
from jax.experimental.pallas import tpu as pltpu

N_SPLIT = 4


def _pool_kernel(*refs, hb):
    o_ref = refs[-1]
    w2 = o_ref.shape[2]
    ho = hb // 2
    for j, x_ref in enumerate(refs[:-1]):
        a = x_ref[0, pl.ds(0, ho, 2), pl.ds(0, w2, 2), :]
        b = x_ref[0, pl.ds(0, ho, 2), pl.ds(1, w2, 2), :]
        c = x_ref[0, pl.ds(1, ho, 2), pl.ds(0, w2, 2), :]
        d = x_ref[0, pl.ds(1, ho, 2), pl.ds(1, w2, 2), :]
        o_ref[0, pl.ds(j * ho, ho)] = (a + b + c + d) * 0.5


def kernel(inputs):
    b, h, w, c = inputs.shape
    h2, w2 = h // 2, w // 2

    hb = 16
    n = N_SPLIT
    while (2 * h2) % (n * hb):
        hb //= 2
    hsteps = (2 * h2) // (n * hb)

    in_specs = [
        pl.BlockSpec(
            (1, hb, w, c),
            functools.partial(lambda j, bi, hi: (bi, hi * n + j, 0, 0), j),
        )
        for j in range(n)
    ]

    return pl.pallas_call(
        functools.partial(_pool_kernel, hb=hb),
        grid=(b, hsteps),
        in_specs=in_specs,
        out_specs=pl.BlockSpec(
            (1, n * hb // 2, w2, c), lambda bi, hi: (bi, hi, 0, 0)
        ),
        out_shape=jax.ShapeDtypeStruct((b, h2, w2, c), inputs.dtype),
        compiler_params=pltpu.CompilerParams(
            dimension_semantics=(pltpu.PARALLEL, pltpu.ARBITRARY),
        ),
    )(*([inputs] * n))


# speedup vs baseline: 9.5091x; 7.7915x over previous
import functools

import jax
import jax.numpy as jnp
from jax.experimental import pallas as pl
from jax.experimental.pallas import tpu as pltpu


def _pool_kernel(x_ref, p_ref, o_ref, *, hb):
    c = x_ref.shape[2]
    w2 = o_ref.shape[3]
    x = x_ref[0].reshape(hb, 2, c, 2 * w2)
    s = x[:, 0] + x[:, 1]
    s2 = s.reshape(hb * c, 2 * w2)
    y = jax.lax.dot(s2, p_ref[...], preferred_element_type=jnp.float32)
    o_ref[0] = y.reshape(hb, c, w2)


def kernel(inputs):
    b, h, w, c = inputs.shape
    h2, w2 = h // 2, w // 2

    xt = inputs.transpose(0, 1, 3, 2)
    pair = jnp.repeat(jnp.eye(w2, dtype=inputs.dtype) * 0.5, 2, axis=0)

    hb = 16
    while h2 % hb:
        hb //= 2

    out = pl.pallas_call(
        functools.partial(_pool_kernel, hb=hb),
        grid=(b, h2 // hb),
        in_specs=[
            pl.BlockSpec((1, 2 * hb, c, w), lambda bi, hi: (bi, hi, 0, 0)),
            pl.BlockSpec((w, w2), lambda bi, hi: (0, 0)),
        ],
        out_specs=pl.BlockSpec((1, hb, c, w2), lambda bi, hi: (bi, hi, 0, 0)),
        out_shape=jax.ShapeDtypeStruct((b, h2, c, w2), inputs.dtype),
        compiler_params=pltpu.CompilerParams(
            dimension_semantics=(pltpu.PARALLEL, pltpu.ARBITRARY),
        ),
    )(xt, pair)
    return out.transpose(0, 1, 3, 2)


# speedup vs baseline: 13.9472x; 1.4667x over previous
import functools

import jax
import jax.numpy as jnp
from jax.experimental import pallas as pl
from jax.experimental.pallas import tpu as pltpu


def _pool_kernel(x_ref, p_ref, o_ref, *, hb):
    c = x_ref.shape[2]
    w2 = o_ref.shape[3]
    x = x_ref[0].reshape(hb, 2, c, 2 * w2)
    s = x[:, 0] + x[:, 1]
    s2 = s.reshape(hb * c, 2 * w2)
    y = jax.lax.dot(s2, p_ref[...], preferred_element_type=jnp.float32)
    o_ref[0] = y.reshape(hb, c, w2)


def kernel(inputs):
    b, h, w, c = inputs.shape
    h2, w2 = h // 2, w // 2

    xt = inputs.transpose(0, 1, 3, 2)
    pair = jnp.repeat(jnp.eye(w2, dtype=inputs.dtype) * 0.5, 2, axis=0)

    hb = 64
    while h2 % hb:
        hb //= 2

    out = pl.pallas_call(
        functools.partial(_pool_kernel, hb=hb),
        grid=(b, h2 // hb),
        in_specs=[
            pl.BlockSpec((1, 2 * hb, c, w), lambda bi, hi: (bi, hi, 0, 0)),
            pl.BlockSpec((w, w2), lambda bi, hi: (0, 0)),
        ],
        out_specs=pl.BlockSpec((1, hb, c, w2), lambda bi, hi: (bi, hi, 0, 0)),
        out_shape=jax.ShapeDtypeStruct((b, h2, c, w2), inputs.dtype),
        compiler_params=pltpu.CompilerParams(
            dimension_semantics=(pltpu.PARALLEL, pltpu.ARBITRARY),
        ),
    )(xt, pair)
    return out.transpose(0, 1, 3, 2)


# speedup vs baseline: 13.9538x; 1.0005x over previous
import functools

import jax
import jax.numpy as jnp
from jax.experimental import pallas as pl
from jax.experimental.pallas import tpu as pltpu


def _pool_kernel(x_ref, p_ref, o_ref, *, hb):
    c = x_ref.shape[2]
    w2 = o_ref.shape[3]
    x = x_ref[0].reshape(hb, 2, c, 2 * w2)
    s = x[:, 0] + x[:, 1]
    s2 = s.reshape(hb * c, 2 * w2)
    y = jax.lax.dot(s2, p_ref[...], preferred_element_type=jnp.float32)
    o_ref[0] = y.reshape(hb, c, w2)


def kernel(inputs):
    b, h, w, c = inputs.shape
    h2, w2 = h // 2, w // 2

    xt = inputs.transpose(0, 1, 3, 2)
    pair = jnp.repeat(jnp.eye(w2, dtype=inputs.dtype) * 0.5, 2, axis=0)

    hb = 128
    while h2 % hb:
        hb //= 2

    out = pl.pallas_call(
        functools.partial(_pool_kernel, hb=hb),
        grid=(b, h2 // hb),
        in_specs=[
            pl.BlockSpec((1, 2 * hb, c, w), lambda bi, hi: (bi, hi, 0, 0)),
            pl.BlockSpec((w, w2), lambda bi, hi: (0, 0)),
        ],
        out_specs=pl.BlockSpec((1, hb, c, w2), lambda bi, hi: (bi, hi, 0, 0)),
        out_shape=jax.ShapeDtypeStruct((b, h2, c, w2), inputs.dtype),
        compiler_params=pltpu.CompilerParams(
            dimension_semantics=(pltpu.PARALLEL, pltpu.ARBITRARY),
        ),
    )(xt, pair)
    return out.transpose(0, 1, 3, 2)
